# Initial kernel scaffold; baseline (speedup 1.0000x reference)
import jax
import jax.numpy as jnp
from jax.experimental import pallas as pl


def kernel(features, edge_index, W1, b1, W2, b2):
    raise NotImplementedError("write your pallas kernel here")


# speedup vs baseline: 4.0868x; 4.0868x over previous
import functools

import jax
import jax.numpy as jnp
from jax import lax
from jax.experimental import pallas as pl
from jax.experimental.pallas import tpu as pltpu
from jax.experimental.pallas import tpu_sc as plsc

_N = 10000
_NP = 10112
_F = 128
_E = 320000
_NW = 32
_K = 128
_EPT = 10112
_EPAD = _EPT * _NW
_RPS = _NP // 16

_mesh = plsc.VectorSubcoreMesh(core_axis_name="c", subcore_axis_name="s")


@functools.partial(
    pl.kernel,
    out_type=(jax.ShapeDtypeStruct((_NW, _NP), jnp.float32),
              jax.ShapeDtypeStruct((_NW, _NP), jnp.float32)),
    mesh=_mesh,
    scratch_types=(
        pltpu.VMEM((_EPT,), jnp.int32),
        pltpu.VMEM((_EPT,), jnp.int32),
        pltpu.VMEM((_NP,), jnp.float32),
        pltpu.VMEM((_NP,), jnp.float32),
    ),
    compiler_params=pltpu.CompilerParams(needs_layout_passes=False),
)
def _degrees(src_hbm, dst_hbm, out_o, out_i, src_v, dst_v, hist_o, hist_i):
    c = lax.axis_index("c")
    s = lax.axis_index("s")
    wid = s * 2 + c

    zero16 = jnp.zeros((16,), jnp.float32)

    def zbody(j, carry):
        hist_o[pl.ds(j * 16, 16)] = zero16
        hist_i[pl.ds(j * 16, 16)] = zero16
        return carry

    lax.fori_loop(0, _NP // 16, zbody, 0)

    pltpu.sync_copy(src_hbm.at[pl.ds(wid * _EPT, _EPT)], src_v)
    pltpu.sync_copy(dst_hbm.at[pl.ds(wid * _EPT, _EPT)], dst_v)

    one16 = jnp.ones((16,), jnp.float32)

    def body(j, carry):
        sl = pl.ds(j * 16, 16)
        plsc.addupdate_scatter(hist_o, [src_v[sl]], one16)
        plsc.addupdate_scatter(hist_i, [dst_v[sl]], one16)
        return carry

    lax.fori_loop(0, _EPT // 16, body, 0)

    pltpu.sync_copy(hist_o, out_o.at[wid])
    pltpu.sync_copy(hist_i, out_i.at[wid])


@functools.partial(
    pl.kernel,
    out_type=jax.ShapeDtypeStruct((2, _NP, _F), jnp.float32),
    mesh=_mesh,
    scratch_types=(
        pltpu.VMEM((_K,), jnp.int32),
        pltpu.VMEM((_K,), jnp.int32),
        pltpu.VMEM((_K, _F), jnp.float32),
        pltpu.VMEM_SHARED((_NP, _F), jnp.float32),
        pltpu.SemaphoreType.DMA,
    ),
)
def _aggregate(hn_hbm, src_hbm, dst_hbm, zeros_hbm, out_hbm,
               idx_s, idx_d, rows, acc, sem):
    c = lax.axis_index("c")
    s = lax.axis_index("s")
    wid = s * 2 + c

    pltpu.sync_copy(zeros_hbm, acc.at[pl.ds(s * _RPS, _RPS)])
    plsc.subcore_barrier()

    base0 = wid * _EPT

    def chunk(i, carry):
        base = base0 + i * _K
        pltpu.sync_copy(src_hbm.at[pl.ds(base, _K)], idx_s)
        pltpu.sync_copy(dst_hbm.at[pl.ds(base, _K)], idx_d)
        pltpu.async_copy(hn_hbm.at[idx_s], rows, sem).wait()
        pltpu.sync_copy(rows, acc.at[idx_d], add=True)
        return carry

    lax.fori_loop(0, _EPT // _K, chunk, 0)

    plsc.subcore_barrier()
    pltpu.sync_copy(acc.at[pl.ds(s * _RPS, _RPS)],
                    out_hbm.at[c, pl.ds(s * _RPS, _RPS)])


def _norms_body(ho_ref, hi_ref, ns_ref, nd_ref):
    dego = jnp.sum(ho_ref[...], axis=0, keepdims=True)
    degi = jnp.sum(hi_ref[...], axis=0, keepdims=True)
    ns_ref[...] = jnp.where(dego > 0, lax.rsqrt(jnp.maximum(dego, 1.0)), 0.0)
    nd_ref[...] = jnp.where(degi > 0, lax.rsqrt(jnp.maximum(degi, 1.0)), 0.0)


_norms = pl.pallas_call(
    _norms_body,
    out_shape=(jax.ShapeDtypeStruct((1, _NP), jnp.float32),
               jax.ShapeDtypeStruct((1, _NP), jnp.float32)),
)

_R = 2528


def _scale_body(x_ref, n_ref, o_ref):
    o_ref[...] = x_ref[...] * n_ref[...]


_scale = pl.pallas_call(
    _scale_body,
    grid=(_NP // _R,),
    in_specs=[pl.BlockSpec((_R, _F), lambda i: (i, 0)),
              pl.BlockSpec((_R, 1), lambda i: (i, 0))],
    out_specs=pl.BlockSpec((_R, _F), lambda i: (i, 0)),
    out_shape=jax.ShapeDtypeStruct((_NP, _F), jnp.float32),
)


def _mm_relu_body(agg_ref, w_ref, b_ref, nd_ref, ns_ref, o_ref):
    agg = agg_ref[0] + agg_ref[1]
    y = jnp.dot(agg, w_ref[...], preferred_element_type=jnp.float32)
    y = y * nd_ref[...] + b_ref[...]
    o_ref[...] = jnp.maximum(y, 0.0) * ns_ref[...]


_mm_relu = pl.pallas_call(
    _mm_relu_body,
    grid=(_NP // _R,),
    in_specs=[pl.BlockSpec((2, _R, _F), lambda i: (0, i, 0)),
              pl.BlockSpec((_F, _F), lambda i: (0, 0)),
              pl.BlockSpec((1, _F), lambda i: (0, 0)),
              pl.BlockSpec((_R, 1), lambda i: (i, 0)),
              pl.BlockSpec((_R, 1), lambda i: (i, 0))],
    out_specs=pl.BlockSpec((_R, _F), lambda i: (i, 0)),
    out_shape=jax.ShapeDtypeStruct((_NP, _F), jnp.float32),
)


def _mm_out_body(agg_ref, w_ref, b_ref, nd_ref, o_ref):
    agg = agg_ref[0] + agg_ref[1]
    y = jnp.dot(agg, w_ref[...], preferred_element_type=jnp.float32)
    o_ref[...] = y * nd_ref[...] + b_ref[...]


_mm_out = pl.pallas_call(
    _mm_out_body,
    grid=(_NP // _R,),
    in_specs=[pl.BlockSpec((2, _R, _F), lambda i: (0, i, 0)),
              pl.BlockSpec((_F, _F), lambda i: (0, 0)),
              pl.BlockSpec((1, _F), lambda i: (0, 0)),
              pl.BlockSpec((_R, 1), lambda i: (i, 0))],
    out_specs=pl.BlockSpec((_R, _F), lambda i: (i, 0)),
    out_shape=jax.ShapeDtypeStruct((_NP, _F), jnp.float32),
)


def kernel(features, edge_index, W1, b1, W2, b2):
    src = edge_index[0].astype(jnp.int32)
    dst = edge_index[1].astype(jnp.int32)
    pad = jnp.full((_EPAD - _E,), _N, jnp.int32)
    src_p = jnp.concatenate([src, pad])
    dst_p = jnp.concatenate([dst, pad])
    feat_p = jnp.concatenate(
        [features.astype(jnp.float32), jnp.zeros((_NP - _N, _F), jnp.float32)])
    zeros = jnp.zeros((_RPS, _F), jnp.float32)

    hist_o, hist_i = _degrees(src_p, dst_p)
    ns_row, nd_row = _norms(hist_o, hist_i)
    ns = ns_row.reshape(_NP, 1)
    nd = nd_row.reshape(_NP, 1)

    hn1 = _scale(feat_p, ns)
    agg1 = _aggregate(hn1, src_p, dst_p, zeros)
    h1n = _mm_relu(agg1, W1, b1.reshape(1, _F), nd, ns)
    agg2 = _aggregate(h1n, src_p, dst_p, zeros)
    out = _mm_out(agg2, W2, b2.reshape(1, _F), nd)
    return out[:_N]


# speedup vs baseline: 4.1629x; 1.0186x over previous
import functools

import jax
import jax.numpy as jnp
from jax import lax
from jax.experimental import pallas as pl
from jax.experimental.pallas import tpu as pltpu
from jax.experimental.pallas import tpu_sc as plsc

_N = 10000
_NP = 10112
_F = 128
_E = 320000
_NW = 32
_K = 128
_EPT = 10240
_EPAD = _EPT * _NW
_RPS = _NP // 16

_mesh = plsc.VectorSubcoreMesh(core_axis_name="c", subcore_axis_name="s")


@functools.partial(
    pl.kernel,
    out_type=(jax.ShapeDtypeStruct((_NW, _NP), jnp.float32),
              jax.ShapeDtypeStruct((_NW, _NP), jnp.float32)),
    mesh=_mesh,
    scratch_types=(
        pltpu.VMEM((_EPT,), jnp.int32),
        pltpu.VMEM((_EPT,), jnp.int32),
        pltpu.VMEM((_NP,), jnp.float32),
        pltpu.VMEM((_NP,), jnp.float32),
    ),
    compiler_params=pltpu.CompilerParams(needs_layout_passes=False),
)
def _degrees(src_hbm, dst_hbm, out_o, out_i, src_v, dst_v, hist_o, hist_i):
    c = lax.axis_index("c")
    s = lax.axis_index("s")
    wid = s * 2 + c

    zero16 = jnp.zeros((16,), jnp.float32)

    def zbody(j, carry):
        hist_o[pl.ds(j * 16, 16)] = zero16
        hist_i[pl.ds(j * 16, 16)] = zero16
        return carry

    lax.fori_loop(0, _NP // 16, zbody, 0)

    pltpu.sync_copy(src_hbm.at[pl.ds(wid * _EPT, _EPT)], src_v)
    pltpu.sync_copy(dst_hbm.at[pl.ds(wid * _EPT, _EPT)], dst_v)

    one16 = jnp.ones((16,), jnp.float32)

    def body(j, carry):
        sl = pl.ds(j * 16, 16)
        plsc.addupdate_scatter(hist_o, [src_v[sl]], one16)
        plsc.addupdate_scatter(hist_i, [dst_v[sl]], one16)
        return carry

    lax.fori_loop(0, _EPT // 16, body, 0)

    pltpu.sync_copy(hist_o, out_o.at[wid])
    pltpu.sync_copy(hist_i, out_i.at[wid])


_NCH = _EPT // _K
_NPASS = 2
_NCHP = _NCH // _NPASS
_EPP = _EPT // _NPASS
_NBUF = 2


@functools.partial(
    pl.kernel,
    out_type=jax.ShapeDtypeStruct((2, _NP, _F), jnp.float32),
    mesh=_mesh,
    scratch_types=(
        pltpu.VMEM((_EPP,), jnp.int32),
        pltpu.VMEM((_NCHP, _K), jnp.int32),
        pltpu.VMEM((_NBUF, _K, _F), jnp.float32),
        pltpu.VMEM_SHARED((_NP, _F), jnp.float32),
        pltpu.SemaphoreType.DMA((_NBUF,)),
    ),
)
def _aggregate(hn_hbm, src_hbm, dst3_hbm, zeros_hbm, out_hbm,
               idx_s, idx_d, rows, acc, sems):
    c = lax.axis_index("c")
    s = lax.axis_index("s")
    wid = s * 2 + c

    pltpu.sync_copy(zeros_hbm, acc.at[pl.ds(s * _RPS, _RPS)])
    plsc.subcore_barrier()

    def gather_start(i, b):
        pltpu.async_copy(hn_hbm.at[idx_s.at[pl.ds(i * _K, _K)]],
                         rows.at[b], sems.at[b])

    for p in range(_NPASS):
        pltpu.sync_copy(
            src_hbm.at[pl.ds(wid * _EPT + p * _EPP, _EPP)], idx_s)
        pltpu.sync_copy(dst3_hbm.at[wid, pl.ds(p * _NCHP, _NCHP)], idx_d)

        gather_start(0, 0)

        def chunk(i, carry):
            b = lax.rem(i, _NBUF)
            nxt = i + 1

            @pl.when(nxt < _NCHP)
            def _():
                gather_start(nxt, lax.rem(nxt, _NBUF))

            pltpu.make_async_copy(hn_hbm.at[idx_s.at[pl.ds(i * _K, _K)]],
                                  rows.at[b], sems.at[b]).wait()
            pltpu.sync_copy(rows.at[b], acc.at[idx_d.at[i]], add=True)
            return carry

        lax.fori_loop(0, _NCHP, chunk, 0)

    plsc.subcore_barrier()
    pltpu.sync_copy(acc.at[pl.ds(s * _RPS, _RPS)],
                    out_hbm.at[c, pl.ds(s * _RPS, _RPS)])


def _norms_body(ho_ref, hi_ref, ns_ref, nd_ref):
    dego = jnp.sum(ho_ref[...], axis=0, keepdims=True)
    degi = jnp.sum(hi_ref[...], axis=0, keepdims=True)
    ns_ref[...] = jnp.where(dego > 0, lax.rsqrt(jnp.maximum(dego, 1.0)), 0.0)
    nd_ref[...] = jnp.where(degi > 0, lax.rsqrt(jnp.maximum(degi, 1.0)), 0.0)


_norms = pl.pallas_call(
    _norms_body,
    out_shape=(jax.ShapeDtypeStruct((1, _NP), jnp.float32),
               jax.ShapeDtypeStruct((1, _NP), jnp.float32)),
)

_R = 2528


def _scale_body(x_ref, n_ref, o_ref):
    o_ref[...] = x_ref[...] * n_ref[...]


_scale = pl.pallas_call(
    _scale_body,
    grid=(_NP // _R,),
    in_specs=[pl.BlockSpec((_R, _F), lambda i: (i, 0)),
              pl.BlockSpec((_R, 1), lambda i: (i, 0))],
    out_specs=pl.BlockSpec((_R, _F), lambda i: (i, 0)),
    out_shape=jax.ShapeDtypeStruct((_NP, _F), jnp.float32),
)


def _mm_relu_body(agg_ref, w_ref, b_ref, nd_ref, ns_ref, o_ref):
    agg = agg_ref[0] + agg_ref[1]
    y = jnp.dot(agg, w_ref[...], preferred_element_type=jnp.float32)
    y = y * nd_ref[...] + b_ref[...]
    o_ref[...] = jnp.maximum(y, 0.0) * ns_ref[...]


_mm_relu = pl.pallas_call(
    _mm_relu_body,
    grid=(_NP // _R,),
    in_specs=[pl.BlockSpec((2, _R, _F), lambda i: (0, i, 0)),
              pl.BlockSpec((_F, _F), lambda i: (0, 0)),
              pl.BlockSpec((1, _F), lambda i: (0, 0)),
              pl.BlockSpec((_R, 1), lambda i: (i, 0)),
              pl.BlockSpec((_R, 1), lambda i: (i, 0))],
    out_specs=pl.BlockSpec((_R, _F), lambda i: (i, 0)),
    out_shape=jax.ShapeDtypeStruct((_NP, _F), jnp.float32),
)


def _mm_out_body(agg_ref, w_ref, b_ref, nd_ref, o_ref):
    agg = agg_ref[0] + agg_ref[1]
    y = jnp.dot(agg, w_ref[...], preferred_element_type=jnp.float32)
    o_ref[...] = y * nd_ref[...] + b_ref[...]


_mm_out = pl.pallas_call(
    _mm_out_body,
    grid=(_NP // _R,),
    in_specs=[pl.BlockSpec((2, _R, _F), lambda i: (0, i, 0)),
              pl.BlockSpec((_F, _F), lambda i: (0, 0)),
              pl.BlockSpec((1, _F), lambda i: (0, 0)),
              pl.BlockSpec((_R, 1), lambda i: (i, 0))],
    out_specs=pl.BlockSpec((_R, _F), lambda i: (i, 0)),
    out_shape=jax.ShapeDtypeStruct((_NP, _F), jnp.float32),
)


def kernel(features, edge_index, W1, b1, W2, b2):
    src = edge_index[0].astype(jnp.int32)
    dst = edge_index[1].astype(jnp.int32)
    pad = jnp.full((_EPAD - _E,), _N, jnp.int32)
    src_p = jnp.concatenate([src, pad])
    dst_p = jnp.concatenate([dst, pad])
    dst3 = dst_p.reshape(_NW, _NCH, _K)
    feat_p = jnp.concatenate(
        [features.astype(jnp.float32), jnp.zeros((_NP - _N, _F), jnp.float32)])
    zeros = jnp.zeros((_RPS, _F), jnp.float32)

    hist_o, hist_i = _degrees(src_p, dst_p)
    ns_row, nd_row = _norms(hist_o, hist_i)
    ns = ns_row.reshape(_NP, 1)
    nd = nd_row.reshape(_NP, 1)

    hn1 = _scale(feat_p, ns)
    agg1 = _aggregate(hn1, src_p, dst3, zeros)
    h1n = _mm_relu(agg1, W1, b1.reshape(1, _F), nd, ns)
    agg2 = _aggregate(h1n, src_p, dst3, zeros)
    out = _mm_out(agg2, W2, b2.reshape(1, _F), nd)
    return out[:_N]
